# Initial kernel scaffold; baseline (speedup 1.0000x reference)
import jax
import jax.numpy as jnp
from jax.experimental import pallas as pl


def kernel(fea, res, queues):
    raise NotImplementedError("write your pallas kernel here")


# speedup vs baseline: 2.3154x; 2.3154x over previous
import jax
import jax.numpy as jnp
from jax.experimental import pallas as pl
from jax.experimental.pallas import tpu as pltpu

INNER = 256
NC = 19
QL = 2975
TEMP = 0.2


def _seg_kernel(res_ref, fea_ref, sums_ref, cnts_ref):
    b = pl.program_id(0)
    res = res_ref[0]
    fea = fea_ref[0]
    hw = res.shape[1]
    maxv = jnp.max(res, axis=0, keepdims=True)
    iota = jax.lax.broadcasted_iota(jnp.int32, (NC, hw), 0)
    idx = jnp.where(res == maxv, iota, NC)
    pred = jnp.min(idx, axis=0, keepdims=True)
    onehot = (iota == pred).astype(jnp.float32)
    part = jax.lax.dot_general(
        onehot, fea, (((1,), (1,)), ((), ())),
        preferred_element_type=jnp.float32,
        precision=jax.lax.Precision.HIGHEST)
    pcnt = jnp.sum(onehot, axis=1, keepdims=True)

    @pl.when(b == 0)
    def _():
        sums_ref[...] = jnp.zeros_like(sums_ref)
        cnts_ref[...] = jnp.zeros_like(cnts_ref)

    sums_ref[...] += part
    cnts_ref[...] += pcnt


def _loss_kernel(sums_ref, cnts_ref, q_ref, loss_ref, stot_ref):
    ph = pl.program_id(0)
    c = pl.program_id(1)

    @pl.when((ph == 0) & (c == 0))
    def _():
        stot_ref[...] = jnp.zeros_like(stot_ref)
        loss_ref[...] = jnp.zeros_like(loss_ref)

    @pl.when(ph == 0)
    def _():
        stot_ref[...] += q_ref[0]

    @pl.when(ph == 1)
    def _():
        q = q_ref[0]
        cnt = cnts_ref[0]
        key = sums_ref[0] / jnp.maximum(cnt, 1.0)
        nrm = jnp.sqrt(jnp.sum(key * key, axis=1, keepdims=True))
        key = key / jnp.maximum(nrm, 1e-12)
        r = jax.lax.broadcasted_iota(jnp.int32, (INNER, INNER), 0)
        col = jax.lax.broadcasted_iota(jnp.int32, (INNER, INNER), 1)
        eye = (r == col).astype(jnp.float32)
        keyT = jax.lax.dot_general(
            eye, key, (((1,), (1,)), ((), ())),
            preferred_element_type=jnp.float32,
            precision=jax.lax.Precision.HIGHEST)
        a = keyT * (1.0 / TEMP)
        xpos = a * q
        xneg = a * (stot_ref[...] - q)
        m = jnp.maximum(jnp.max(xpos, axis=1, keepdims=True),
                        jnp.max(xneg, axis=1, keepdims=True))
        s = (jnp.sum(jnp.exp(xpos - m), axis=1, keepdims=True) +
             jnp.sum(jnp.exp(xneg - m), axis=1, keepdims=True))
        lse = m + jnp.log(s)
        loss_c = jnp.sum(lse - xpos[:, 0:1], axis=0, keepdims=True) / INNER
        loss_ref[...] += jnp.where(cnt > 0.0, loss_c, 0.0)


def kernel(fea, res, queues):
    bs = fea.shape[0]
    hw = fea.shape[2] * fea.shape[3]
    fea_r = fea.reshape(bs, INNER, hw)
    res_r = res.reshape(bs, NC, hw)

    sums, cnts = pl.pallas_call(
        _seg_kernel,
        grid=(bs,),
        in_specs=[
            pl.BlockSpec((1, NC, hw), lambda b: (b, 0, 0)),
            pl.BlockSpec((1, INNER, hw), lambda b: (b, 0, 0)),
        ],
        out_specs=[
            pl.BlockSpec((NC, INNER), lambda b: (0, 0)),
            pl.BlockSpec((NC, 1), lambda b: (0, 0)),
        ],
        out_shape=[
            jax.ShapeDtypeStruct((NC, INNER), jnp.float32),
            jax.ShapeDtypeStruct((NC, 1), jnp.float32),
        ],
    )(res_r, fea_r)

    sums3 = sums.reshape(NC, 1, INNER)
    cnts3 = cnts.reshape(NC, 1, 1)

    loss = pl.pallas_call(
        _loss_kernel,
        grid=(2, NC),
        in_specs=[
            pl.BlockSpec((1, 1, INNER), lambda ph, c: (c, 0, 0)),
            pl.BlockSpec((1, 1, 1), lambda ph, c: (c, 0, 0)),
            pl.BlockSpec((1, INNER, QL), lambda ph, c: (c, 0, 0)),
        ],
        out_specs=pl.BlockSpec((1, 1), lambda ph, c: (0, 0)),
        out_shape=jax.ShapeDtypeStruct((1, 1), jnp.float32),
        scratch_shapes=[pltpu.VMEM((INNER, QL), jnp.float32)],
    )(sums3, cnts3, queues)

    return (res, loss[0, 0])


# speedup vs baseline: 2.6463x; 1.1429x over previous
import jax
import jax.numpy as jnp
from jax.experimental import pallas as pl
from jax.experimental.pallas import tpu as pltpu

INNER = 256
NC = 19
QL = 2975
TEMP = 0.2


def _seg_kernel(res_ref, fea_ref, sums_ref, cnts_ref):
    b = pl.program_id(0)
    res = res_ref[0]
    fea = fea_ref[0]
    hw = res.shape[1]
    maxv = jnp.max(res, axis=0, keepdims=True)
    iota = jax.lax.broadcasted_iota(jnp.int32, (NC, hw), 0)
    idx = jnp.where(res == maxv, iota, NC)
    pred = jnp.min(idx, axis=0, keepdims=True)
    onehot = (iota == pred).astype(jnp.float32)
    part = jax.lax.dot_general(
        onehot, fea, (((1,), (1,)), ((), ())),
        preferred_element_type=jnp.float32,
        precision=jax.lax.Precision.DEFAULT)
    pcnt = jnp.sum(onehot, axis=1, keepdims=True)

    @pl.when(b == 0)
    def _():
        sums_ref[...] = jnp.zeros_like(sums_ref)
        cnts_ref[...] = jnp.zeros_like(cnts_ref)

    sums_ref[...] += part
    cnts_ref[...] += pcnt


def _loss_kernel(sums_ref, cnts_ref, q_ref, loss_ref, stot_ref, qcache_ref):
    ph = pl.program_id(0)
    c = pl.program_id(1)

    @pl.when((ph == 0) & (c == 0))
    def _():
        stot_ref[...] = jnp.zeros_like(stot_ref)
        loss_ref[...] = jnp.zeros_like(loss_ref)

    @pl.when(ph == 0)
    def _():
        q0 = q_ref[0]
        stot_ref[...] += q0
        qcache_ref[c] = q0.astype(jnp.bfloat16)

    @pl.when(ph == 1)
    def _():
        q = qcache_ref[c].astype(jnp.float32)
        cnt = cnts_ref[0]
        key = sums_ref[0] / jnp.maximum(cnt, 1.0)
        nrm = jnp.sqrt(jnp.sum(key * key, axis=1, keepdims=True))
        key = key / jnp.maximum(nrm, 1e-12)
        r = jax.lax.broadcasted_iota(jnp.int32, (INNER, INNER), 0)
        col = jax.lax.broadcasted_iota(jnp.int32, (INNER, INNER), 1)
        eye = (r == col).astype(jnp.float32)
        keyT = jax.lax.dot_general(
            eye, key, (((1,), (1,)), ((), ())),
            preferred_element_type=jnp.float32,
            precision=jax.lax.Precision.HIGHEST)
        a = keyT * (1.0 / TEMP)
        xpos = a * q
        xneg = a * (stot_ref[...] - q)
        m = jnp.maximum(jnp.max(xpos, axis=1, keepdims=True),
                        jnp.max(xneg, axis=1, keepdims=True))
        s = (jnp.sum(jnp.exp(xpos - m), axis=1, keepdims=True) +
             jnp.sum(jnp.exp(xneg - m), axis=1, keepdims=True))
        lse = m + jnp.log(s)
        loss_c = jnp.sum(lse - xpos[:, 0:1], axis=0, keepdims=True) / INNER
        loss_ref[...] += jnp.where(cnt > 0.0, loss_c, 0.0)


def kernel(fea, res, queues):
    bs = fea.shape[0]
    hw = fea.shape[2] * fea.shape[3]
    fea_r = fea.reshape(bs, INNER, hw)
    res_r = res.reshape(bs, NC, hw)

    sums, cnts = pl.pallas_call(
        _seg_kernel,
        grid=(bs,),
        in_specs=[
            pl.BlockSpec((1, NC, hw), lambda b: (b, 0, 0)),
            pl.BlockSpec((1, INNER, hw), lambda b: (b, 0, 0)),
        ],
        out_specs=[
            pl.BlockSpec((NC, INNER), lambda b: (0, 0)),
            pl.BlockSpec((NC, 1), lambda b: (0, 0)),
        ],
        out_shape=[
            jax.ShapeDtypeStruct((NC, INNER), jnp.float32),
            jax.ShapeDtypeStruct((NC, 1), jnp.float32),
        ],
    )(res_r, fea_r)

    sums3 = sums.reshape(NC, 1, INNER)
    cnts3 = cnts.reshape(NC, 1, 1)

    loss = pl.pallas_call(
        _loss_kernel,
        grid=(2, NC),
        in_specs=[
            pl.BlockSpec((1, 1, INNER), lambda ph, c: (c, 0, 0)),
            pl.BlockSpec((1, 1, 1), lambda ph, c: (c, 0, 0)),
            pl.BlockSpec((1, INNER, QL), lambda ph, c: (c * (1 - ph), 0, 0)),
        ],
        out_specs=pl.BlockSpec((1, 1), lambda ph, c: (0, 0)),
        out_shape=jax.ShapeDtypeStruct((1, 1), jnp.float32),
        scratch_shapes=[
            pltpu.VMEM((INNER, QL), jnp.float32),
            pltpu.VMEM((NC, INNER, QL), jnp.bfloat16),
        ],
    )(sums3, cnts3, queues)

    return (res, loss[0, 0])


# speedup vs baseline: 2.7760x; 1.0490x over previous
import jax
import jax.numpy as jnp
from jax.experimental import pallas as pl
from jax.experimental.pallas import tpu as pltpu

INNER = 256
NC = 19
QL = 2975
TEMP = 0.2


def _seg_kernel(res_ref, fea_ref, sums_ref, cnts_ref):
    b = pl.program_id(0)
    h = pl.program_id(1)
    res = res_ref[0]
    fea = fea_ref[0]
    hw = res.shape[1]
    maxv = jnp.max(res, axis=0, keepdims=True)
    iota = jax.lax.broadcasted_iota(jnp.int32, (NC, hw), 0)
    idx = jnp.where(res == maxv, iota, NC)
    pred = jnp.min(idx, axis=0, keepdims=True)
    onehot = (iota == pred).astype(jnp.float32)
    part = jax.lax.dot_general(
        onehot, fea, (((1,), (1,)), ((), ())),
        preferred_element_type=jnp.float32,
        precision=jax.lax.Precision.DEFAULT)
    pcnt = jnp.sum(onehot, axis=1, keepdims=True)

    @pl.when((b == 0) & (h == 0))
    def _():
        sums_ref[...] = jnp.zeros_like(sums_ref)
        cnts_ref[...] = jnp.zeros_like(cnts_ref)

    sums_ref[...] += part
    cnts_ref[...] += pcnt


def _key_vec(sums_row, cnt):
    key = sums_row / jnp.maximum(cnt, 1.0)
    nrm = jnp.sqrt(jnp.sum(key * key, axis=1, keepdims=True))
    key = key / jnp.maximum(nrm, 1e-12)
    r = jax.lax.broadcasted_iota(jnp.int32, (INNER, INNER), 0)
    col = jax.lax.broadcasted_iota(jnp.int32, (INNER, INNER), 1)
    eye = (r == col).astype(jnp.float32)
    keyT = jax.lax.dot_general(
        eye, key, (((1,), (1,)), ((), ())),
        preferred_element_type=jnp.float32,
        precision=jax.lax.Precision.HIGHEST)
    return keyT * (1.0 / TEMP)


def _loss_kernel(sums_ref, cnts_ref, q_ref, loss_ref,
                 stot_ref, qcache_ref, mpos_ref, spos_ref, x0_ref):
    ph = pl.program_id(0)
    c = pl.program_id(1)

    @pl.when((ph == 0) & (c == 0))
    def _():
        stot_ref[...] = jnp.zeros_like(stot_ref)
        loss_ref[...] = jnp.zeros_like(loss_ref)

    @pl.when(ph == 0)
    def _():
        q0 = q_ref[0]
        stot_ref[...] += q0
        qcache_ref[c] = q0.astype(jnp.bfloat16)
        a = _key_vec(sums_ref[0], cnts_ref[0])
        xp = a * q0
        mp = jnp.max(xp, axis=1, keepdims=True)
        sp = jnp.sum(jnp.exp(xp - mp), axis=1, keepdims=True)
        mpos_ref[c] = mp
        spos_ref[c] = sp
        x0_ref[c] = xp[:, 0:1]

    @pl.when(ph == 1)
    def _():
        q = qcache_ref[c].astype(jnp.float32)
        a = _key_vec(sums_ref[0], cnts_ref[0])
        xn = a * (stot_ref[...] - q)
        mn = jnp.max(xn, axis=1, keepdims=True)
        sn = jnp.sum(jnp.exp(xn - mn), axis=1, keepdims=True)
        mp = mpos_ref[c]
        sp = spos_ref[c]
        m = jnp.maximum(mp, mn)
        s = sp * jnp.exp(mp - m) + sn * jnp.exp(mn - m)
        lse = m + jnp.log(s)
        loss_c = jnp.sum(lse - x0_ref[c], axis=0, keepdims=True) / INNER
        cnt = cnts_ref[0]
        loss_ref[...] += jnp.where(cnt > 0.0, loss_c[0:1, :], 0.0)


def kernel(fea, res, queues):
    bs = fea.shape[0]
    hw = fea.shape[2] * fea.shape[3]
    hw2 = hw // 2
    fea_r = fea.reshape(bs, INNER, hw)
    res_r = res.reshape(bs, NC, hw)

    sums, cnts = pl.pallas_call(
        _seg_kernel,
        grid=(bs, 2),
        in_specs=[
            pl.BlockSpec((1, NC, hw2), lambda b, h: (b, 0, h)),
            pl.BlockSpec((1, INNER, hw2), lambda b, h: (b, 0, h)),
        ],
        out_specs=[
            pl.BlockSpec((NC, INNER), lambda b, h: (0, 0)),
            pl.BlockSpec((NC, 1), lambda b, h: (0, 0)),
        ],
        out_shape=[
            jax.ShapeDtypeStruct((NC, INNER), jnp.float32),
            jax.ShapeDtypeStruct((NC, 1), jnp.float32),
        ],
    )(res_r, fea_r)

    sums3 = sums.reshape(NC, 1, INNER)
    cnts3 = cnts.reshape(NC, 1, 1)

    loss = pl.pallas_call(
        _loss_kernel,
        grid=(2, NC),
        in_specs=[
            pl.BlockSpec((1, 1, INNER), lambda ph, c: (c, 0, 0)),
            pl.BlockSpec((1, 1, 1), lambda ph, c: (c, 0, 0)),
            pl.BlockSpec((1, INNER, QL), lambda ph, c: (c * (1 - ph), 0, 0)),
        ],
        out_specs=pl.BlockSpec((1, 1), lambda ph, c: (0, 0)),
        out_shape=jax.ShapeDtypeStruct((1, 1), jnp.float32),
        scratch_shapes=[
            pltpu.VMEM((INNER, QL), jnp.float32),
            pltpu.VMEM((NC, INNER, QL), jnp.bfloat16),
            pltpu.VMEM((NC, INNER, 1), jnp.float32),
            pltpu.VMEM((NC, INNER, 1), jnp.float32),
            pltpu.VMEM((NC, INNER, 1), jnp.float32),
        ],
    )(sums3, cnts3, queues)

    return (res, loss[0, 0])
